# Initial kernel scaffold; baseline (speedup 1.0000x reference)
import jax
import jax.numpy as jnp
from jax.experimental import pallas as pl


def kernel(x, edge_index, edge_type, emb, W1, root1, b1, W2, root2, b2):
    raise NotImplementedError("write your pallas kernel here")


# speedup vs baseline: 1.6525x; 1.6525x over previous
import functools

import jax
import jax.numpy as jnp
from jax import lax
from jax.experimental import pallas as pl
from jax.experimental.pallas import tpu as pltpu
from jax.experimental.pallas import tpu_sc as plsc

N = 10000
R = 8
H = 128
E = 320000

NC = 2
NS = 16
NP = 10240
DUMP = N
ROWS_PT = NP // NS

TCH = E // (NC * NS)
SUB = 2000
NSUB = TCH // SUB
VPS = SUB // 16
CAP = 10240
KB = 128
CW = 8
BLK = 640

_mesh = plsc.VectorSubcoreMesh(core_axis_name="c", subcore_axis_name="s")


@functools.partial(
    pl.kernel,
    out_type=(
        jax.ShapeDtypeStruct((NC * NS * R * CAP,), jnp.int32),
        jax.ShapeDtypeStruct((NC * NS * R * CAP,), jnp.int32),
        jax.ShapeDtypeStruct((NC * NS * 16,), jnp.int32),
    ),
    mesh=_mesh,
    compiler_params=pltpu.CompilerParams(needs_layout_passes=False),
    scratch_types=[
        pltpu.VMEM((SUB,), jnp.int32),
        pltpu.VMEM((SUB,), jnp.int32),
        pltpu.VMEM((SUB,), jnp.int32),
        pltpu.VMEM((R * (SUB + 16),), jnp.int32),
        pltpu.VMEM((R * (SUB + 16),), jnp.int32),
        pltpu.VMEM((KB,), jnp.int32),
        pltpu.VMEM((KB,), jnp.int32),
        pltpu.VMEM((16,), jnp.int32),
    ],
)
def _partition(es_hbm, ed_hbm, et_hbm, psrc_hbm, pdst_hbm, nblk_hbm,
               src_ch, dst_ch, rel_ch, sbuf, dbuf, dmy_s, dmy_d, nbv):
    c = lax.axis_index("c")
    s = lax.axis_index("s")
    wid = c * NS + s
    base_e = wid * TCH
    seg0 = wid * (R * CAP)

    zero16 = jnp.zeros((16,), jnp.int32)
    dump16 = jnp.full((16,), DUMP, jnp.int32)
    for j in range(KB // 16):
        dmy_s[pl.ds(j * 16, 16)] = zero16
        dmy_d[pl.ds(j * 16, 16)] = dump16

    ptrs = [jnp.int32(0)] * R
    for scn in range(NSUB):
        eb = base_e + scn * SUB
        pltpu.sync_copy(es_hbm.at[pl.ds(eb, SUB)], src_ch)
        pltpu.sync_copy(ed_hbm.at[pl.ds(eb, SUB)], dst_ch)
        pltpu.sync_copy(et_hbm.at[pl.ds(eb, SUB)], rel_ch)

        def body(j, lp):
            sv = src_ch[pl.ds(j * 16, 16)]
            dv = dst_ch[pl.ds(j * 16, 16)]
            rv = rel_ch[pl.ds(j * 16, 16)]
            out = []
            for r in range(R):
                m = rv == r
                plsc.store_compressed(
                    sbuf.at[pl.ds(r * (SUB + 16) + lp[r], 16)], sv, mask=m)
                plsc.store_compressed(
                    dbuf.at[pl.ds(r * (SUB + 16) + lp[r], 16)], dv, mask=m)
                cnt = plsc.all_reduce_population_count(m)
                out.append(lp[r] + cnt[0])
            return tuple(out)

        lp = lax.fori_loop(0, VPS, body, tuple([jnp.int32(0)] * R))
        for r in range(R):
            sbuf[pl.ds(r * (SUB + 16) + lp[r], 16)] = zero16
            dbuf[pl.ds(r * (SUB + 16) + lp[r], 16)] = dump16
            pltpu.sync_copy(sbuf.at[pl.ds(r * (SUB + 16), SUB + 16)],
                            psrc_hbm.at[pl.ds(seg0 + r * CAP + ptrs[r], SUB + 16)])
            pltpu.sync_copy(dbuf.at[pl.ds(r * (SUB + 16), SUB + 16)],
                            pdst_hbm.at[pl.ds(seg0 + r * CAP + ptrs[r], SUB + 16)])
            ptrs[r] = ptrs[r] + (lp[r] + 15) // 16 * 16

    lane = lax.iota(jnp.int32, 16)
    nv = jnp.zeros((16,), jnp.int32)
    for r in range(R):
        pltpu.sync_copy(dmy_s, psrc_hbm.at[pl.ds(seg0 + r * CAP + ptrs[r], KB)])
        pltpu.sync_copy(dmy_d, pdst_hbm.at[pl.ds(seg0 + r * CAP + ptrs[r], KB)])
        nv = jnp.where(lane == r, (ptrs[r] + KB - 1) // KB, nv)
    nbv[...] = nv
    pltpu.sync_copy(nbv, nblk_hbm.at[pl.ds(wid * 16, 16)])


@functools.partial(
    pl.kernel,
    out_type=jax.ShapeDtypeStruct((R, NP, H), jnp.float32),
    mesh=_mesh,
    compiler_params=pltpu.CompilerParams(needs_layout_passes=False),
    scratch_types=[
        pltpu.VMEM((KB,), jnp.int32),
        pltpu.VMEM((KB,), jnp.int32),
        pltpu.VMEM((KB, H), jnp.float32),
        pltpu.VMEM((16,), jnp.int32),
        pltpu.VMEM_SHARED((NP, H), jnp.float32),
        pltpu.SemaphoreType.DMA,
    ],
)
def _scatter(psrc_h, pdst_h, nblk_h, h_h, zrow_h,
             S_h, src_idx, dst_idx, rows, nbl, asp, sem):
    c = lax.axis_index("c")
    s = lax.axis_index("s")
    row0 = s * ROWS_PT
    for ri in range(R // NC):
        r = c * (R // NC) + ri
        pltpu.sync_copy(zrow_h, asp.at[pl.ds(row0, ROWS_PT)])
        plsc.subcore_barrier()
        for creg in range(NC):
            wseg = creg * NS + s
            pltpu.sync_copy(nblk_h.at[pl.ds(wseg * 16, 16)], nbl)
            lane = lax.iota(jnp.int32, 16)
            nb = jnp.sum(jnp.where(lane == r, nbl[...], 0))
            sbase = wseg * (R * CAP) + r * CAP

            def blk(i, carry):
                pltpu.sync_copy(psrc_h.at[pl.ds(sbase + i * KB, KB)],
                                src_idx)
                pltpu.sync_copy(pdst_h.at[pl.ds(sbase + i * KB, KB)],
                                dst_idx)
                pltpu.async_copy(h_h.at[src_idx], rows, sem).wait()
                pltpu.sync_copy(rows, asp.at[dst_idx], add=True)
                return carry

            lax.fori_loop(0, nb, blk, jnp.int32(0))
        plsc.subcore_barrier()
        pltpu.sync_copy(asp.at[pl.ds(row0, ROWS_PT)],
                        S_h.at[r, pl.ds(row0, ROWS_PT)])


@functools.partial(
    pl.kernel,
    out_type=jax.ShapeDtypeStruct((R, NP, H), jnp.float32),
    mesh=_mesh,
    compiler_params=pltpu.CompilerParams(needs_layout_passes=False),
    scratch_types=[
        pltpu.VMEM((KB,), jnp.int32),
        pltpu.VMEM((KB, H), jnp.float32),
        pltpu.VMEM((16,), jnp.int32),
        pltpu.VMEM_SHARED((NP, H), jnp.float32),
    ],
)
def _counts(psrc_h, pdst_h, nblk_h, ones_h, zrow_h,
            C_h, dst_idx, ones_v, nbl, csp):
    c = lax.axis_index("c")
    s = lax.axis_index("s")
    row0 = s * ROWS_PT
    pltpu.sync_copy(ones_h, ones_v)
    for ri in range(R // NC):
        r = c * (R // NC) + ri
        pltpu.sync_copy(zrow_h, csp.at[pl.ds(row0, ROWS_PT)])
        plsc.subcore_barrier()
        for creg in range(NC):
            wseg = creg * NS + s
            pltpu.sync_copy(nblk_h.at[pl.ds(wseg * 16, 16)], nbl)
            lane = lax.iota(jnp.int32, 16)
            nb = jnp.sum(jnp.where(lane == r, nbl[...], 0))
            sbase = wseg * (R * CAP) + r * CAP

            def blk(i, carry):
                pltpu.sync_copy(pdst_h.at[pl.ds(sbase + i * KB, KB)],
                                dst_idx)
                pltpu.sync_copy(ones_v, csp.at[dst_idx], add=True)
                return carry

            lax.fori_loop(0, nb, blk, jnp.int32(0))
        plsc.subcore_barrier()
        pltpu.sync_copy(csp.at[pl.ds(row0, ROWS_PT)],
                        C_h.at[r, pl.ds(row0, ROWS_PT)])


def _tc_layer(h_pad, S, C, W, root, b8):
    def body(h_ref, S_ref, c_ref, W_ref, root_ref, b_ref, o_ref):
        acc = jnp.dot(h_ref[...], root_ref[...],
                      preferred_element_type=jnp.float32)
        for r in range(R):
            inv = 1.0 / jnp.maximum(c_ref[r][:, 0:1], 1.0)
            acc = acc + jnp.dot(S_ref[r] * inv, W_ref[r],
                                preferred_element_type=jnp.float32)
        o_ref[...] = jnp.maximum(acc + b_ref[0:1, :], 0.0)

    return pl.pallas_call(
        body,
        grid=(NP // BLK,),
        in_specs=[
            pl.BlockSpec((BLK, H), lambda i: (i, 0)),
            pl.BlockSpec((R, BLK, H), lambda i: (0, i, 0)),
            pl.BlockSpec((R, BLK, H), lambda i: (0, i, 0)),
            pl.BlockSpec((R, H, H), lambda i: (0, 0, 0)),
            pl.BlockSpec((H, H), lambda i: (0, 0)),
            pl.BlockSpec((8, H), lambda i: (0, 0)),
        ],
        out_specs=pl.BlockSpec((BLK, H), lambda i: (i, 0)),
        out_shape=jax.ShapeDtypeStruct((NP, H), jnp.float32),
    )(h_pad, S, C, W, root, b8)


def kernel(x, edge_index, edge_type, emb, W1, root1, b1, W2, root2, b2):
    h0 = jnp.take(emb, x, axis=0)
    h0 = jnp.concatenate(
        [h0, jnp.zeros((NP - N, H), jnp.float32)], axis=0)

    psrc, pdst, nblk = _partition(edge_index[0], edge_index[1], edge_type)

    zrow = jnp.zeros((ROWS_PT, H), jnp.float32)
    ones = jnp.ones((KB, H), jnp.float32)

    C = _counts(psrc, pdst, nblk, ones, zrow)
    S1 = _scatter(psrc, pdst, nblk, h0, zrow)
    h1 = _tc_layer(h0, S1, C, W1, root1, jnp.broadcast_to(b1, (8, H)))
    S2 = _scatter(psrc, pdst, nblk, h1, zrow)
    h2 = _tc_layer(h1, S2, C, W2, root2, jnp.broadcast_to(b2, (8, H)))
    return h2[:N]


# speedup vs baseline: 1.6641x; 1.0070x over previous
import functools

import jax
import jax.numpy as jnp
from jax import lax
from jax.experimental import pallas as pl
from jax.experimental.pallas import tpu as pltpu
from jax.experimental.pallas import tpu_sc as plsc

N = 10000
R = 8
H = 128
E = 320000

NC = 2
NS = 16
NP = 10240
DUMP = N
ROWS_PT = NP // NS

TCH = E // (NC * NS)
SUB = 2000
NSUB = TCH // SUB
VPS = SUB // 16
CAP = 10240
KB = 128
CW = 8
BLK = 640
DEPTH = 2

_mesh = plsc.VectorSubcoreMesh(core_axis_name="c", subcore_axis_name="s")


@functools.partial(
    pl.kernel,
    out_type=(
        jax.ShapeDtypeStruct((NC * NS * R * CAP,), jnp.int32),
        jax.ShapeDtypeStruct((NC * NS * R * CAP,), jnp.int32),
        jax.ShapeDtypeStruct((NC * NS * 16,), jnp.int32),
    ),
    mesh=_mesh,
    compiler_params=pltpu.CompilerParams(needs_layout_passes=False),
    scratch_types=[
        pltpu.VMEM((SUB,), jnp.int32),
        pltpu.VMEM((SUB,), jnp.int32),
        pltpu.VMEM((SUB,), jnp.int32),
        pltpu.VMEM((R * (SUB + 16),), jnp.int32),
        pltpu.VMEM((R * (SUB + 16),), jnp.int32),
        pltpu.VMEM((KB,), jnp.int32),
        pltpu.VMEM((KB,), jnp.int32),
        pltpu.VMEM((16,), jnp.int32),
    ],
)
def _partition(es_hbm, ed_hbm, et_hbm, psrc_hbm, pdst_hbm, nblk_hbm,
               src_ch, dst_ch, rel_ch, sbuf, dbuf, dmy_s, dmy_d, nbv):
    c = lax.axis_index("c")
    s = lax.axis_index("s")
    wid = c * NS + s
    base_e = wid * TCH
    seg0 = wid * (R * CAP)

    zero16 = jnp.zeros((16,), jnp.int32)
    dump16 = jnp.full((16,), DUMP, jnp.int32)
    for j in range(KB // 16):
        dmy_s[pl.ds(j * 16, 16)] = zero16
        dmy_d[pl.ds(j * 16, 16)] = dump16

    ptrs = [jnp.int32(0)] * R
    for scn in range(NSUB):
        eb = base_e + scn * SUB
        pltpu.sync_copy(es_hbm.at[pl.ds(eb, SUB)], src_ch)
        pltpu.sync_copy(ed_hbm.at[pl.ds(eb, SUB)], dst_ch)
        pltpu.sync_copy(et_hbm.at[pl.ds(eb, SUB)], rel_ch)

        def body(j, lp):
            sv = src_ch[pl.ds(j * 16, 16)]
            dv = dst_ch[pl.ds(j * 16, 16)]
            rv = rel_ch[pl.ds(j * 16, 16)]
            out = []
            for r in range(R):
                m = rv == r
                plsc.store_compressed(
                    sbuf.at[pl.ds(r * (SUB + 16) + lp[r], 16)], sv, mask=m)
                plsc.store_compressed(
                    dbuf.at[pl.ds(r * (SUB + 16) + lp[r], 16)], dv, mask=m)
                cnt = plsc.all_reduce_population_count(m)
                out.append(lp[r] + cnt[0])
            return tuple(out)

        lp = lax.fori_loop(0, VPS, body, tuple([jnp.int32(0)] * R))
        for r in range(R):
            sbuf[pl.ds(r * (SUB + 16) + lp[r], 16)] = zero16
            dbuf[pl.ds(r * (SUB + 16) + lp[r], 16)] = dump16
            pltpu.sync_copy(sbuf.at[pl.ds(r * (SUB + 16), SUB + 16)],
                            psrc_hbm.at[pl.ds(seg0 + r * CAP + ptrs[r], SUB + 16)])
            pltpu.sync_copy(dbuf.at[pl.ds(r * (SUB + 16), SUB + 16)],
                            pdst_hbm.at[pl.ds(seg0 + r * CAP + ptrs[r], SUB + 16)])
            ptrs[r] = ptrs[r] + (lp[r] + 15) // 16 * 16

    lane = lax.iota(jnp.int32, 16)
    nv = jnp.zeros((16,), jnp.int32)
    for r in range(R):
        pltpu.sync_copy(dmy_s, psrc_hbm.at[pl.ds(seg0 + r * CAP + ptrs[r], KB)])
        pltpu.sync_copy(dmy_d, pdst_hbm.at[pl.ds(seg0 + r * CAP + ptrs[r], KB)])
        nv = jnp.where(lane == r, (ptrs[r] + KB - 1) // KB, nv)
    nbv[...] = nv
    pltpu.sync_copy(nbv, nblk_hbm.at[pl.ds(wid * 16, 16)])


@functools.partial(
    pl.kernel,
    out_type=jax.ShapeDtypeStruct((R, NP, H), jnp.float32),
    mesh=_mesh,
    compiler_params=pltpu.CompilerParams(needs_layout_passes=False),
    scratch_types=[
        pltpu.VMEM((DEPTH * KB,), jnp.int32),
        pltpu.VMEM((DEPTH, KB), jnp.int32),
        pltpu.VMEM((DEPTH * KB, H), jnp.float32),
        pltpu.VMEM((16,), jnp.int32),
        pltpu.VMEM_SHARED((NP, H), jnp.float32),
        pltpu.SemaphoreType.DMA,
        pltpu.SemaphoreType.DMA,
    ],
)
def _scatter(psrc_h, pdst_h, nblk_h, h_h, zrow_h,
             S_h, src_idx4, dst_idx2, rows, nbl, asp, sem, isem):
    c = lax.axis_index("c")
    s = lax.axis_index("s")
    row0 = s * ROWS_PT
    for ri in range(R // NC):
        r = c * (R // NC) + ri
        pltpu.sync_copy(zrow_h, asp.at[pl.ds(row0, ROWS_PT)])
        plsc.subcore_barrier()
        for creg in range(NC):
            wseg = creg * NS + s
            pltpu.sync_copy(nblk_h.at[pl.ds(wseg * 16, 16)], nbl)
            lane = lax.iota(jnp.int32, 16)
            nb = jnp.sum(jnp.where(lane == r, nbl[...], 0))
            sbase = wseg * (R * CAP) + r * CAP

            def sblk(k, carry):
                base = sbase + k * (DEPTH * KB)
                idx_ds = [pltpu.async_copy(
                    psrc_h.at[pl.ds(base, DEPTH * KB)], src_idx4, isem)]
                for j in range(DEPTH):
                    idx_ds.append(pltpu.async_copy(
                        pdst_h.at[pl.ds(base + j * KB, KB)],
                        dst_idx2.at[j], isem))
                for d in idx_ds:
                    d.wait()
                g_ds = [pltpu.async_copy(
                    h_h.at[src_idx4.at[pl.ds(j * KB, KB)]],
                    rows.at[pl.ds(j * KB, KB)], sem) for j in range(DEPTH)]
                for d in g_ds:
                    d.wait()
                for j in range(DEPTH):
                    pltpu.sync_copy(rows.at[pl.ds(j * KB, KB)],
                                    asp.at[dst_idx2.at[j]], add=True)
                return carry

            lax.fori_loop(0, nb // DEPTH, sblk, jnp.int32(0))

            def blk(i, carry):
                pltpu.sync_copy(psrc_h.at[pl.ds(sbase + i * KB, KB)],
                                src_idx4.at[pl.ds(0, KB)])
                pltpu.sync_copy(pdst_h.at[pl.ds(sbase + i * KB, KB)],
                                dst_idx2.at[0])
                pltpu.async_copy(
                    h_h.at[src_idx4.at[pl.ds(0, KB)]],
                    rows.at[pl.ds(0, KB)], sem).wait()
                pltpu.sync_copy(rows.at[pl.ds(0, KB)],
                                asp.at[dst_idx2.at[0]], add=True)
                return carry

            lax.fori_loop(nb // DEPTH * DEPTH, nb, blk, jnp.int32(0))
        plsc.subcore_barrier()
        pltpu.sync_copy(asp.at[pl.ds(row0, ROWS_PT)],
                        S_h.at[r, pl.ds(row0, ROWS_PT)])


@functools.partial(
    pl.kernel,
    out_type=jax.ShapeDtypeStruct((R, NP, H), jnp.float32),
    mesh=_mesh,
    compiler_params=pltpu.CompilerParams(needs_layout_passes=False),
    scratch_types=[
        pltpu.VMEM((KB,), jnp.int32),
        pltpu.VMEM((KB, H), jnp.float32),
        pltpu.VMEM((16,), jnp.int32),
        pltpu.VMEM_SHARED((NP, H), jnp.float32),
    ],
)
def _counts(psrc_h, pdst_h, nblk_h, ones_h, zrow_h,
            C_h, dst_idx, ones_v, nbl, csp):
    c = lax.axis_index("c")
    s = lax.axis_index("s")
    row0 = s * ROWS_PT
    pltpu.sync_copy(ones_h, ones_v)
    for ri in range(R // NC):
        r = c * (R // NC) + ri
        pltpu.sync_copy(zrow_h, csp.at[pl.ds(row0, ROWS_PT)])
        plsc.subcore_barrier()
        for creg in range(NC):
            wseg = creg * NS + s
            pltpu.sync_copy(nblk_h.at[pl.ds(wseg * 16, 16)], nbl)
            lane = lax.iota(jnp.int32, 16)
            nb = jnp.sum(jnp.where(lane == r, nbl[...], 0))
            sbase = wseg * (R * CAP) + r * CAP

            def blk(i, carry):
                pltpu.sync_copy(pdst_h.at[pl.ds(sbase + i * KB, KB)],
                                dst_idx)
                pltpu.sync_copy(ones_v, csp.at[dst_idx], add=True)
                return carry

            lax.fori_loop(0, nb, blk, jnp.int32(0))
        plsc.subcore_barrier()
        pltpu.sync_copy(csp.at[pl.ds(row0, ROWS_PT)],
                        C_h.at[r, pl.ds(row0, ROWS_PT)])


def _tc_layer(h_pad, S, C, W, root, b8):
    def body(h_ref, S_ref, c_ref, W_ref, root_ref, b_ref, o_ref):
        acc = jnp.dot(h_ref[...], root_ref[...],
                      preferred_element_type=jnp.float32)
        for r in range(R):
            inv = 1.0 / jnp.maximum(c_ref[r][:, 0:1], 1.0)
            acc = acc + jnp.dot(S_ref[r] * inv, W_ref[r],
                                preferred_element_type=jnp.float32)
        o_ref[...] = jnp.maximum(acc + b_ref[0:1, :], 0.0)

    return pl.pallas_call(
        body,
        grid=(NP // BLK,),
        in_specs=[
            pl.BlockSpec((BLK, H), lambda i: (i, 0)),
            pl.BlockSpec((R, BLK, H), lambda i: (0, i, 0)),
            pl.BlockSpec((R, BLK, H), lambda i: (0, i, 0)),
            pl.BlockSpec((R, H, H), lambda i: (0, 0, 0)),
            pl.BlockSpec((H, H), lambda i: (0, 0)),
            pl.BlockSpec((8, H), lambda i: (0, 0)),
        ],
        out_specs=pl.BlockSpec((BLK, H), lambda i: (i, 0)),
        out_shape=jax.ShapeDtypeStruct((NP, H), jnp.float32),
    )(h_pad, S, C, W, root, b8)


def kernel(x, edge_index, edge_type, emb, W1, root1, b1, W2, root2, b2):
    h0 = jnp.take(emb, x, axis=0)
    h0 = jnp.concatenate(
        [h0, jnp.zeros((NP - N, H), jnp.float32)], axis=0)

    psrc, pdst, nblk = _partition(edge_index[0], edge_index[1], edge_type)

    zrow = jnp.zeros((ROWS_PT, H), jnp.float32)
    ones = jnp.ones((KB, H), jnp.float32)

    C = _counts(psrc, pdst, nblk, ones, zrow)
    S1 = _scatter(psrc, pdst, nblk, h0, zrow)
    h1 = _tc_layer(h0, S1, C, W1, root1, jnp.broadcast_to(b1, (8, H)))
    S2 = _scatter(psrc, pdst, nblk, h1, zrow)
    h2 = _tc_layer(h1, S2, C, W2, root2, jnp.broadcast_to(b2, (8, H)))
    return h2[:N]
